# baseline (speedup 1.0000x reference)
import jax
import jax.numpy as jnp
from jax.experimental import pallas as pl
from jax.experimental.pallas import tpu as pltpu

N_TIME, PER_DIM = 6, 6
HI = jax.lax.Precision.HIGHEST


def _body(inp_ref, dates_ref, cmax_ref, m6_ref, b40_ref, sp_ref, le_ref,
          out_ref, vid_ref, base_ref, valsp_ref):
    d = pl.program_id(1)
    _, L, C = out_ref.shape
    D_IN = inp_ref.shape[2]

    @pl.when(d == 0)
    def _compute_base():
        r3 = jax.lax.broadcasted_iota(jnp.int32, (3, C), 0)
        c3 = jax.lax.broadcasted_iota(jnp.int32, (3, C), 1)
        m3 = (c3 == r3 + (C - 3)).astype(jnp.float32)
        xa = (jnp.dot(dates_ref[0], m6_ref[...],
                      preferred_element_type=jnp.float32, precision=HI)
              + jnp.dot(cmax_ref[0], m3,
                        preferred_element_type=jnp.float32, precision=HI)
              + b40_ref[...])
        c = jax.lax.broadcasted_iota(jnp.int32, (L, C), 1)
        sinsel = (c >= 1) & (c <= N_TIME * PER_DIM) & ((c - 1) % PER_DIM != 0)
        t40 = jnp.where(sinsel, jnp.sin(xa), xa)
        base_ref[...] = le_ref[...] + t40
        valsp_ref[...] = inp_ref[0] + sp_ref[...]

    r = jax.lax.broadcasted_iota(jnp.int32, (D_IN, C), 0)
    cc = jax.lax.broadcasted_iota(jnp.int32, (D_IN, C), 1)
    oneh = ((r == d) & (cc == 0)).astype(jnp.float32)
    out_ref[0] = base_ref[...] + jnp.dot(
        valsp_ref[...], oneh, preferred_element_type=jnp.float32, precision=HI)
    vid_ref[...] = jnp.full((1, 1, 1, L), d, dtype=jnp.int32)


def kernel(input, dates, cmax, time_w, time_b, local_emb, space_emb):
    b, length, d_input = input.shape
    d_model = local_emb.shape[1]
    n_time, per_dim = time_w.shape
    w_flat, b_flat = time_w.reshape(-1), time_b.reshape(-1)
    cc = jnp.arange(d_model)
    valid = (cc >= 1) & (cc <= n_time * per_dim)
    src = jnp.clip(cc - 1, 0, n_time * per_dim - 1)
    b40 = jnp.where(valid, b_flat[src], 0.0)[None, :]
    m6 = jnp.where(
        valid[None, :] & (jnp.arange(n_time)[:, None] == (src // per_dim)),
        w_flat[src][None, :], 0.0)

    out, vid = pl.pallas_call(
        _body,
        grid=(b, d_input),
        in_specs=[
            pl.BlockSpec((1, length, d_input), lambda bb, dd: (bb, 0, 0)),
            pl.BlockSpec((1, length, n_time), lambda bb, dd: (bb, 0, 0)),
            pl.BlockSpec((1, length, 3), lambda bb, dd: (bb, 0, 0)),
            pl.BlockSpec((n_time, d_model), lambda bb, dd: (0, 0)),
            pl.BlockSpec((1, d_model), lambda bb, dd: (0, 0)),
            pl.BlockSpec((1, d_input), lambda bb, dd: (0, 0)),
            pl.BlockSpec((length, d_model), lambda bb, dd: (0, 0)),
        ],
        out_specs=[
            pl.BlockSpec((1, length, d_model), lambda bb, dd: (bb, dd, 0)),
            pl.BlockSpec((1, 1, 1, length), lambda bb, dd: (bb, dd, 0, 0)),
        ],
        out_shape=[
            jax.ShapeDtypeStruct((b, d_input * length, d_model), jnp.float32),
            jax.ShapeDtypeStruct((b, d_input, 1, length), jnp.int32),
        ],
        scratch_shapes=[pltpu.VMEM((length, d_model), jnp.float32),
                        pltpu.VMEM((length, d_input), jnp.float32)],
        compiler_params=pltpu.CompilerParams(
            dimension_semantics=("arbitrary", "arbitrary")),
    )(input, dates, cmax, m6, b40, space_emb.reshape(1, d_input), local_emb)
    return out, vid.reshape(b, d_input * length)


# speedup vs baseline: 1.7212x; 1.7212x over previous
import jax
import jax.numpy as jnp
from jax.experimental import pallas as pl
from jax.experimental.pallas import tpu as pltpu

N_TIME, PER_DIM = 6, 6
LC = 512


def _body(inp_ref, feat_ref, w_ref, b_ref, sp_ref, le_ref, out_ref, vid_ref):
    lc = inp_ref.shape[1]
    c_dim = feat_ref.shape[2]
    d_in = inp_ref.shape[2]
    xa = feat_ref[0] * w_ref[...] + b_ref[...]
    c = jax.lax.broadcasted_iota(jnp.int32, (lc, c_dim), 1)
    sinsel = (c >= 1) & (c <= N_TIME * PER_DIM) & ((c - 1) % PER_DIM != 0)
    base = le_ref[...] + jnp.where(sinsel, jnp.sin(xa), xa)
    valsp = inp_ref[0] + sp_ref[...]
    for dd in range(d_in):
        col = jax.lax.slice(valsp, (0, dd), (lc, dd + 1))
        out_ref[0, dd] = base + jax.lax.pad(col, 0.0, ((0, 0, 0), (0, c_dim - 1, 0)))
    vid_ref[0] = jax.lax.broadcasted_iota(jnp.int32, (d_in, lc), 0)


def kernel(input, dates, cmax, time_w, time_b, local_emb, space_emb):
    b, length, d_input = input.shape
    d_model = local_emb.shape[1]
    n_time, per_dim = time_w.shape
    nt = n_time * per_dim
    feats = jnp.concatenate(
        [jnp.zeros((b, length, 1), jnp.float32),
         jnp.repeat(dates, per_dim, axis=-1), cmax], axis=-1)
    w40 = jnp.concatenate(
        [jnp.zeros((1,), jnp.float32), time_w.reshape(-1),
         jnp.ones((d_model - 1 - nt,), jnp.float32)])[None, :]
    b40 = jnp.concatenate(
        [jnp.zeros((1,), jnp.float32), time_b.reshape(-1),
         jnp.zeros((d_model - 1 - nt,), jnp.float32)])[None, :]

    out, vid = pl.pallas_call(
        _body,
        grid=(b, length // LC),
        in_specs=[
            pl.BlockSpec((1, LC, d_input), lambda bb, ll: (bb, ll, 0)),
            pl.BlockSpec((1, LC, d_model), lambda bb, ll: (bb, ll, 0)),
            pl.BlockSpec((1, d_model), lambda bb, ll: (0, 0)),
            pl.BlockSpec((1, d_model), lambda bb, ll: (0, 0)),
            pl.BlockSpec((1, d_input), lambda bb, ll: (0, 0)),
            pl.BlockSpec((LC, d_model), lambda bb, ll: (ll, 0)),
        ],
        out_specs=[
            pl.BlockSpec((1, d_input, LC, d_model), lambda bb, ll: (bb, 0, ll, 0)),
            pl.BlockSpec((1, d_input, LC), lambda bb, ll: (bb, 0, ll)),
        ],
        out_shape=[
            jax.ShapeDtypeStruct((b, d_input, length, d_model), jnp.float32),
            jax.ShapeDtypeStruct((b, d_input, length), jnp.int32),
        ],
        compiler_params=pltpu.CompilerParams(
            dimension_semantics=("arbitrary", "arbitrary")),
    )(input, feats, w40, b40, space_emb.reshape(1, d_input), local_emb)
    return (out.reshape(b, d_input * length, d_model),
            vid.reshape(b, d_input * length))
